# baseline (speedup 1.0000x reference)
import functools

import jax
import jax.numpy as jnp
from jax import lax
from jax.experimental import pallas as pl
from jax.experimental.pallas import tpu as pltpu
from jax.experimental.pallas import tpu_sc as plsc

_NUM_EMB = 1001
_EMB_DIM = 64
_BATCH = 16384

_INFO = plsc.get_sparse_core_info()
_NC = _INFO.num_cores
_NS = _INFO.num_subcores
_NW = _NC * _NS
_L = _INFO.num_lanes

_NDIMG = 8
_NBATG = _NW // _NDIMG
_ROWS = _EMB_DIM // _NDIMG
_BCOLS = _BATCH // _NBATG
_TPAD = 1008
_NTILES = _BCOLS // 128

_mesh = plsc.VectorSubcoreMesh(core_axis_name="c", subcore_axis_name="s")


@functools.partial(
    pl.kernel,
    mesh=_mesh,
    out_type=jax.ShapeDtypeStruct((_EMB_DIM, _BATCH), jnp.float32),
    scratch_types=[
        pltpu.VMEM((_ROWS * _TPAD,), jnp.float32),
        pltpu.VMEM((_BCOLS,), jnp.int32),
        pltpu.VMEM((_ROWS, 128), jnp.float32),
        pltpu.VMEM((_ROWS, 128), jnp.float32),
        pltpu.SemaphoreType.DMA,
        pltpu.SemaphoreType.DMA,
    ],
    compiler_params=pltpu.CompilerParams(
        use_tc_tiling_on_sc=True, needs_layout_passes=False
    ),
)
def _sc_gather_t(idx_hbm, tflat_hbm, out_hbm, tv, iv, tile_a, tile_b, sem_a, sem_b):
    wid = lax.axis_index("s") * _NC + lax.axis_index("c")
    g = wid % _NDIMG
    b = wid // _NDIMG
    pltpu.sync_copy(tflat_hbm.at[pl.ds(g * _ROWS * _TPAD, _ROWS * _TPAD)], tv)
    pltpu.sync_copy(idx_hbm.at[pl.ds(b * _BCOLS, _BCOLS)], iv)

    def build(tile, t):
        for c in range(128 // _L):
            ivec = iv[pl.ds(t * 128 + c * _L, _L)]
            addrs = [ivec] + [ivec + d * _TPAD for d in range(1, _ROWS)]
            vals = [plsc.load_gather(tv, [a]) for a in addrs]
            for d in range(_ROWS):
                tile[d, pl.ds(c * _L, _L)] = vals[d]

    def out_slice(t):
        return out_hbm.at[pl.ds(g * _ROWS, _ROWS), pl.ds(b * _BCOLS + t * 128, 128)]

    build(tile_a, 0)

    def body(i, carry):
        t0 = 2 * i
        wa = pltpu.async_copy(tile_a, out_slice(t0), sem_a)
        build(tile_b, t0 + 1)
        wa.wait()
        wb = pltpu.async_copy(tile_b, out_slice(t0 + 1), sem_b)

        @pl.when(i < _NTILES // 2 - 1)
        def _():
            build(tile_a, t0 + 2)

        wb.wait()
        return carry

    lax.fori_loop(0, _NTILES // 2, body, jnp.int32(0))


def kernel(user_ids, table):
    tflat = jnp.pad(table.T, ((0, 0), (0, _TPAD - _NUM_EMB))).reshape(-1)
    out_t = _sc_gather_t(user_ids, tflat)
    return jnp.transpose(out_t)
